# baseline (speedup 1.0000x reference)
import functools

import jax
import jax.numpy as jnp
from jax.experimental import pallas as pl

_E = 8
_INNER = 8
_BSZ = 32
_OUT = _INNER * _BSZ
_HID = 1024
_TOK = 2048
_TILE_T = 256

_pallas_call = pl.pallas_call


def _router_body(x_ref, w_ref, w01_ref, e01_ref):
    x = x_ref[...]
    w = w_ref[...]
    logits = jax.lax.dot_general(
        x, w, (((1,), (1,)), ((), ())), preferred_element_type=jnp.float32)
    m = jnp.max(logits, axis=-1, keepdims=True)
    ex = jnp.exp(logits - m)
    sm = ex / jnp.sum(ex, axis=-1, keepdims=True)
    iot = jax.lax.broadcasted_iota(jnp.int32, sm.shape, 1)
    v0 = jnp.max(sm, axis=-1, keepdims=True)
    a0 = jnp.min(jnp.where(sm == v0, iot, _E), axis=-1, keepdims=True)
    sm1 = jnp.where(iot == a0, -1.0, sm)
    v1 = jnp.max(sm1, axis=-1, keepdims=True)
    a1 = jnp.min(jnp.where(sm1 == v1, iot, _E), axis=-1, keepdims=True)
    s = v0 + v1
    w01_ref[...] = jnp.where(iot == 0, v0 / s, jnp.where(iot == 1, v1 / s, 0.0))
    e01_ref[...] = jnp.where(iot == 0, a0, jnp.where(iot == 1, a1, 0))


def _scores_body(x_ref, wg_ref, wu_ref, avg_ref, s_ref):
    x = x_ref[...]
    wg = wg_ref[0]
    wu = wu_ref[0]
    g = jax.lax.dot_general(
        x, wg, (((1,), (1,)), ((), ())), preferred_element_type=jnp.float32)
    u = jax.lax.dot_general(
        x, wu, (((1,), (1,)), ((), ())), preferred_element_type=jnp.float32)
    p = jnp.abs(u * g * jax.nn.sigmoid(g))
    s_ref[0] = jax.lax.dot_general(
        p, avg_ref[...], (((1,), (0,)), ((), ())),
        preferred_element_type=jnp.float32)


def _select_body(s_ref, e01_ref, w01_ref, fw_ref, fid_ref):
    e01 = e01_ref[...]
    w01 = w01_ref[...]
    n = e01.shape[0]
    e0 = e01[:, 0:1]
    e1 = e01[:, 1:2]
    w0 = w01[:, 0:1]
    w1 = w01[:, 1:2]
    s0 = jnp.zeros((n, _INNER), jnp.float32)
    s1 = jnp.zeros((n, _INNER), jnp.float32)
    for e in range(_E):
        se = s_ref[e]
        s0 = jnp.where(e0 == e, se, s0)
        s1 = jnp.where(e1 == e, se, s1)
    jot = jax.lax.broadcasted_iota(jnp.int32, (n, _INNER), 1)

    def ranks(s):
        r = jnp.zeros((n, _INNER), jnp.int32)
        for jp in range(_INNER):
            c = s[:, jp:jp + 1]
            beat = (c > s) | ((c == s) & (jp < jot))
            r = r + beat.astype(jnp.int32)
        return r

    def desc_ids(sel, k):
        pos = jnp.zeros((n, _INNER), jnp.int32)
        for jp in range(_INNER):
            pos = pos + (sel[:, jp:jp + 1] & (jp > jot)).astype(jnp.int32)
        cols = []
        for m in range(k):
            hit = sel & (pos == m)
            cols.append(jnp.sum(jnp.where(hit, jot, 0), axis=1, keepdims=True))
        return jnp.concatenate(cols, axis=1)

    sel0 = ranks(s0) < 4
    sel1 = ranks(s1) < 2
    i0 = desc_ids(sel0, 4) + e0 * _INNER
    i1 = desc_ids(sel1, 2) + e1 * _INNER
    ids_a = jnp.concatenate([i0, i1], axis=1)
    ids_b = jnp.concatenate([i1, i0], axis=1)
    fid_ref[...] = jnp.where(e0 > e1, ids_a, ids_b)
    fw_ref[...] = jnp.concatenate(
        [jnp.broadcast_to(w0, (n, 4)), jnp.broadcast_to(w1, (n, 2))], axis=1)


def kernel(hidden_states, gating_output, topk, renormalize, out_gate_weight,
           stacked_in_gate_weights, stacked_in_up_weights):
    del gating_output, topk, renormalize
    x = hidden_states.astype(jnp.float32)
    n = x.shape[0]

    w01, e01 = _pallas_call(
        _router_body,
        out_shape=(
            jax.ShapeDtypeStruct((n, _E), jnp.float32),
            jax.ShapeDtypeStruct((n, _E), jnp.int32),
        ),
    )(x, out_gate_weight.astype(jnp.float32))

    avg = (jnp.equal(
        jnp.arange(_OUT)[:, None] // _BSZ,
        jnp.arange(_INNER)[None, :]).astype(jnp.float32) / _BSZ)

    n_t = n // _TILE_T
    scores = _pallas_call(
        _scores_body,
        grid=(_E, n_t),
        in_specs=[
            pl.BlockSpec((_TILE_T, _HID), lambda e, t: (t, 0)),
            pl.BlockSpec((1, _OUT, _HID), lambda e, t: (e, 0, 0)),
            pl.BlockSpec((1, _OUT, _HID), lambda e, t: (e, 0, 0)),
            pl.BlockSpec((_OUT, _INNER), lambda e, t: (0, 0)),
        ],
        out_specs=pl.BlockSpec((1, _TILE_T, _INNER), lambda e, t: (e, t, 0)),
        out_shape=jax.ShapeDtypeStruct((_E, n, _INNER), jnp.float32),
    )(x.astype(jnp.bfloat16),
      stacked_in_gate_weights.astype(jnp.bfloat16),
      stacked_in_up_weights.astype(jnp.bfloat16), avg)

    fw, fid = _pallas_call(
        _select_body,
        out_shape=(
            jax.ShapeDtypeStruct((n, 6), jnp.float32),
            jax.ShapeDtypeStruct((n, 6), jnp.int32),
        ),
    )(scores, e01, w01)
    return fw, fid


# speedup vs baseline: 1.0702x; 1.0702x over previous
import functools

import jax
import jax.numpy as jnp
from jax import lax
from jax.experimental import pallas as pl
from jax.experimental.pallas import tpu as pltpu
from jax.experimental.pallas import tpu_sc as plsc

_E = 8
_INNER = 8
_BSZ = 32
_OUT = _INNER * _BSZ
_HID = 1024
_TOK = 2048
_TILE = 128
_NT = 40
_NP = _NT * _TILE
_NW = 32
_RPW = _NP // _NW
_CH = 32
_NCH = _RPW // _CH

_pallas_call = pl.pallas_call


def _router_body(x_ref, w_ref, w01_ref, d01_ref, et_ref):
    x = x_ref[...]
    w = w_ref[...]
    logits = jax.lax.dot_general(
        x, w, (((1,), (1,)), ((), ())), preferred_element_type=jnp.float32)
    m = jnp.max(logits, axis=-1, keepdims=True)
    ex = jnp.exp(logits - m)
    sm = ex / jnp.sum(ex, axis=-1, keepdims=True)
    iot = jax.lax.broadcasted_iota(jnp.int32, sm.shape, 1)
    v0 = jnp.max(sm, axis=-1, keepdims=True)
    a0 = jnp.min(jnp.where(sm == v0, iot, _E), axis=-1, keepdims=True)
    sm1 = jnp.where(iot == a0, -1.0, sm)
    v1 = jnp.max(sm1, axis=-1, keepdims=True)
    a1 = jnp.min(jnp.where(sm1 == v1, iot, _E), axis=-1, keepdims=True)
    s = v0 + v1
    w01_ref[...] = jnp.where(iot == 0, v0 / s, jnp.where(iot == 1, v1 / s, 0.0))
    cmat = ((a0 == iot).astype(jnp.float32) + (a1 == iot).astype(jnp.float32))
    counts = jnp.sum(cmat, axis=0, keepdims=True)
    padded = jnp.ceil(counts / _TILE) * _TILE
    base = jnp.zeros((1, _E), jnp.float32)
    for j in range(_E - 1):
        base = base + jnp.where(iot[:1, :] > j, padded[:, j:j + 1], 0.0)
    bi = jax.lax.broadcasted_iota(jnp.int32, (_TILE, _TILE), 0)
    bj = jax.lax.broadcasted_iota(jnp.int32, (_TILE, _TILE), 1)
    tril = (bj < bi).astype(jnp.float32)
    running = jnp.zeros((1, _E), jnp.float32)
    for b in range(x.shape[0] // _TILE):
        lo = b * _TILE
        cb = cmat[lo:lo + _TILE, :]
        pexc = jax.lax.dot_general(
            tril, cb, (((1,), (0,)), ((), ())),
            preferred_element_type=jnp.float32) + running
        running = running + jnp.sum(cb, axis=0, keepdims=True)
        a0b = a0[lo:lo + _TILE, :]
        a1b = a1[lo:lo + _TILE, :]
        d0 = jnp.zeros((_TILE, 1), jnp.float32)
        d1 = jnp.zeros((_TILE, 1), jnp.float32)
        for e in range(_E):
            be = base[:, e:e + 1]
            d0 = d0 + jnp.where(a0b == e, be + pexc[:, e:e + 1], 0.0)
            d1 = d1 + jnp.where(a1b == e, be + pexc[:, e:e + 1], 0.0)
        d01_ref[lo:lo + _TILE, :] = jnp.concatenate(
            [d0.astype(jnp.int32), d1.astype(jnp.int32)], axis=1)
    ti = (jax.lax.broadcasted_iota(jnp.int32, (1, _NT), 1)
          .astype(jnp.float32) * _TILE)
    etf = jnp.zeros((1, _NT), jnp.float32)
    for e in range(_E):
        etf = etf + (ti >= base[:, e:e + 1]).astype(jnp.float32)
    et_ref[...] = etf.astype(jnp.int32) - 1


def _sc_dispatch(x, d0, d1):
    mesh = plsc.VectorSubcoreMesh(core_axis_name="c", subcore_axis_name="s")

    @functools.partial(
        pl.kernel, mesh=mesh,
        out_type=jax.ShapeDtypeStruct((_NP, _HID), jnp.float32),
        scratch_types=[
            pltpu.VMEM((32,), jnp.int32),
            pltpu.VMEM((32,), jnp.int32),
            pltpu.VMEM((32,), jnp.int32),
            pltpu.VMEM((32,), jnp.int32),
            pltpu.VMEM((32, _HID), jnp.float32),
            pltpu.VMEM((32, _HID), jnp.float32),
            pltpu.SemaphoreType.DMA,
            pltpu.SemaphoreType.DMA,
            pltpu.SemaphoreType.DMA,
            pltpu.SemaphoreType.DMA,
        ],
    )
    def k(x_hbm, d0_hbm, d1_hbm, xs_hbm, de0, do0, de1, do1, rows0, rows1,
          s00, s01, s10, s11):
        wid = lax.axis_index("s") * 2 + lax.axis_index("c")
        tok0 = wid * 64
        de = (de0, de1)
        do = (do0, do1)
        rows = (rows0, rows1)
        sa = (s00, s10)
        sb = (s01, s11)
        copies = []
        for c in range(2):
            pltpu.sync_copy(d0_hbm.at[pl.ds(tok0 + 32 * c, 32)], de[c])
            pltpu.sync_copy(d1_hbm.at[pl.ds(tok0 + 32 * c, 32)], do[c])
            pltpu.sync_copy(x_hbm.at[pl.ds(tok0 + 32 * c, 32)], rows[c])
            copies.append(pltpu.async_copy(rows[c], xs_hbm.at[de[c]], sa[c]))
            copies.append(pltpu.async_copy(rows[c], xs_hbm.at[do[c]], sb[c]))
        for cp in copies:
            cp.wait()

    return k(x, d0, d1)


def _gmm_body(et_ref, xs_ref, wg_ref, wu_ref, avg_ref, pk_ref):
    i = pl.program_id(0)
    e = et_ref[i]
    x = xs_ref[...]
    wg = wg_ref[pl.ds(e, 1)][0]
    wu = wu_ref[pl.ds(e, 1)][0]
    g = jax.lax.dot_general(
        x, wg, (((1,), (1,)), ((), ())), preferred_element_type=jnp.float32)
    u = jax.lax.dot_general(
        x, wu, (((1,), (1,)), ((), ())), preferred_element_type=jnp.float32)
    p = jnp.abs(u * g * jax.nn.sigmoid(g))
    n = p.shape[0]
    s = jax.lax.dot_general(
        p, avg_ref[...], (((1,), (0,)), ((), ())),
        preferred_element_type=jnp.float32)
    jot = jax.lax.broadcasted_iota(jnp.int32, (n, _INNER), 1)
    r = jnp.zeros((n, _INNER), jnp.int32)
    for jp in range(_INNER):
        c = s[:, jp:jp + 1]
        r = r + ((c > s) | ((c == s) & (jp < jot))).astype(jnp.int32)
    pk = jnp.full((n, 1), e * 4096, jnp.int32)
    for rank in range(4):
        t_r = jnp.sum(jnp.where(r == rank, jot, 0), axis=1, keepdims=True)
        pk = pk + t_r * (8 ** rank)
    pk_ref[...] = jnp.broadcast_to(pk, (n, 128))


def _sc_gather_words(pk, d0, d1):
    mesh = plsc.VectorSubcoreMesh(core_axis_name="c", subcore_axis_name="s")
    tpw = _TOK // _NW

    @functools.partial(
        pl.kernel, mesh=mesh,
        out_type=(jax.ShapeDtypeStruct((_TOK, 128), jnp.int32),
                  jax.ShapeDtypeStruct((_TOK, 128), jnp.int32)),
        scratch_types=[
            pltpu.VMEM((tpw,), jnp.int32),
            pltpu.VMEM((tpw,), jnp.int32),
            pltpu.VMEM((tpw, 128), jnp.int32),
            pltpu.VMEM((tpw, 128), jnp.int32),
            pltpu.SemaphoreType.DMA,
            pltpu.SemaphoreType.DMA,
        ],
    )
    def k(pk_hbm, d0_hbm, d1_hbm, o0_hbm, o1_hbm, i0, i1, r0, r1, s0, s1):
        wid = lax.axis_index("s") * 2 + lax.axis_index("c")
        base = wid * tpw
        pltpu.sync_copy(d0_hbm.at[pl.ds(base, tpw)], i0)
        pltpu.sync_copy(d1_hbm.at[pl.ds(base, tpw)], i1)
        c0 = pltpu.async_copy(pk_hbm.at[i0], r0, s0)
        c1 = pltpu.async_copy(pk_hbm.at[i1], r1, s1)
        c0.wait()
        pltpu.sync_copy(r0, o0_hbm.at[pl.ds(base, tpw)])
        c1.wait()
        pltpu.sync_copy(r1, o1_hbm.at[pl.ds(base, tpw)])

    return k(pk, d0, d1)


def _select_body(p0_ref, p1_ref, w01_ref, fw_ref, fid_ref):
    pk0 = p0_ref[:, 0:1]
    pk1 = p1_ref[:, 0:1]
    w01 = w01_ref[...]
    n = pk0.shape[0]
    e0 = pk0 >> 12
    e1 = pk1 >> 12
    t = [(pk0 >> (3 * rr)) & 7 for rr in range(4)]
    for (a, b) in ((0, 1), (2, 3), (0, 2), (1, 3), (1, 2)):
        hi = jnp.maximum(t[a], t[b])
        lo = jnp.minimum(t[a], t[b])
        t[a], t[b] = hi, lo
    ids0 = [x + e0 * _INNER for x in t]
    s1a = pk1 & 7
    s1b = (pk1 >> 3) & 7
    ids1 = [jnp.maximum(s1a, s1b) + e1 * _INNER,
            jnp.minimum(s1a, s1b) + e1 * _INNER]
    ids_a = jnp.concatenate(ids0 + ids1, axis=1)
    ids_b = jnp.concatenate(ids1 + ids0, axis=1)
    fid_ref[...] = jnp.where(e0 > e1, ids_a, ids_b)
    w0 = w01[:, 0:1]
    w1 = w01[:, 1:2]
    fw_ref[...] = jnp.concatenate(
        [jnp.broadcast_to(w0, (n, 4)), jnp.broadcast_to(w1, (n, 2))], axis=1)


def kernel(hidden_states, gating_output, topk, renormalize, out_gate_weight,
           stacked_in_gate_weights, stacked_in_up_weights):
    del gating_output, topk, renormalize
    x = hidden_states.astype(jnp.float32)
    n = x.shape[0]

    w01, d01, et2 = _pallas_call(
        _router_body,
        out_shape=(
            jax.ShapeDtypeStruct((n, _E), jnp.float32),
            jax.ShapeDtypeStruct((n, 2), jnp.int32),
            jax.ShapeDtypeStruct((1, _NT), jnp.int32),
        ),
    )(x, out_gate_weight.astype(jnp.float32))

    d0 = d01[:, 0].reshape(-1)
    d1 = d01[:, 1].reshape(-1)
    xs = _sc_dispatch(x, d0, d1)

    avg = (jnp.equal(
        jnp.arange(_OUT)[:, None] // _BSZ,
        jnp.arange(_INNER)[None, :]).astype(jnp.float32) / _BSZ)

    pk = _pallas_call(
        _gmm_body,
        grid_spec=pltpu.PrefetchScalarGridSpec(
            num_scalar_prefetch=1,
            grid=(_NT,),
            in_specs=[
                pl.BlockSpec((_TILE, _HID), lambda i, et: (i, 0)),
                pl.BlockSpec((_E, _OUT, _HID), lambda i, et: (0, 0, 0)),
                pl.BlockSpec((_E, _OUT, _HID), lambda i, et: (0, 0, 0)),
                pl.BlockSpec((_OUT, _INNER), lambda i, et: (0, 0)),
            ],
            out_specs=pl.BlockSpec((_TILE, 128), lambda i, et: (i, 0)),
        ),
        out_shape=jax.ShapeDtypeStruct((_NP, 128), jnp.int32),
    )(et2.reshape(-1), xs, stacked_in_gate_weights, stacked_in_up_weights, avg)

    p0, p1 = _sc_gather_words(pk, d0, d1)

    fw, fid = _pallas_call(
        _select_body,
        out_shape=(
            jax.ShapeDtypeStruct((n, 6), jnp.float32),
            jax.ShapeDtypeStruct((n, 6), jnp.int32),
        ),
    )(p0, p1, w01)
    return fw, fid


# speedup vs baseline: 1.1355x; 1.0610x over previous
import functools

import jax
import jax.numpy as jnp
from jax import lax
from jax.experimental import pallas as pl
from jax.experimental.pallas import tpu as pltpu
from jax.experimental.pallas import tpu_sc as plsc

_E = 8
_INNER = 8
_BSZ = 32
_OUT = _INNER * _BSZ
_HID = 1024
_TOK = 2048
_TILE = 128
_NT = 40
_NP = _NT * _TILE
_NW = 32
_RPW = _NP // _NW
_CH = 32
_NCH = _RPW // _CH

_pallas_call = pl.pallas_call


def _router_body(x_ref, w_ref, w01_ref, d01_ref, et_ref):
    x = x_ref[...]
    w = w_ref[...]
    logits = jax.lax.dot_general(
        x, w, (((1,), (1,)), ((), ())), preferred_element_type=jnp.float32)
    m = jnp.max(logits, axis=-1, keepdims=True)
    ex = jnp.exp(logits - m)
    sm = ex / jnp.sum(ex, axis=-1, keepdims=True)
    iot = jax.lax.broadcasted_iota(jnp.int32, sm.shape, 1)
    v0 = jnp.max(sm, axis=-1, keepdims=True)
    a0 = jnp.min(jnp.where(sm == v0, iot, _E), axis=-1, keepdims=True)
    sm1 = jnp.where(iot == a0, -1.0, sm)
    v1 = jnp.max(sm1, axis=-1, keepdims=True)
    a1 = jnp.min(jnp.where(sm1 == v1, iot, _E), axis=-1, keepdims=True)
    s = v0 + v1
    w01_ref[...] = jnp.where(iot == 0, v0 / s, jnp.where(iot == 1, v1 / s, 0.0))
    cmat = ((a0 == iot).astype(jnp.float32) + (a1 == iot).astype(jnp.float32))
    counts = jnp.sum(cmat, axis=0, keepdims=True)
    padded = jnp.ceil(counts / _TILE) * _TILE
    base = jnp.zeros((1, _E), jnp.float32)
    for j in range(_E - 1):
        base = base + jnp.where(iot[:1, :] > j, padded[:, j:j + 1], 0.0)
    bi = jax.lax.broadcasted_iota(jnp.int32, (_TILE, _TILE), 0)
    bj = jax.lax.broadcasted_iota(jnp.int32, (_TILE, _TILE), 1)
    tril = (bj < bi).astype(jnp.float32)
    running = jnp.zeros((1, _E), jnp.float32)
    for b in range(x.shape[0] // _TILE):
        lo = b * _TILE
        cb = cmat[lo:lo + _TILE, :]
        pexc = jax.lax.dot_general(
            tril, cb, (((1,), (0,)), ((), ())),
            preferred_element_type=jnp.float32) + running
        running = running + jnp.sum(cb, axis=0, keepdims=True)
        a0b = a0[lo:lo + _TILE, :]
        a1b = a1[lo:lo + _TILE, :]
        bp = base + pexc
        iotb = iot[:_TILE, :]
        d0 = jnp.sum(jnp.where(a0b == iotb, bp, 0.0), axis=1, keepdims=True)
        d1 = jnp.sum(jnp.where(a1b == iotb, bp, 0.0), axis=1, keepdims=True)
        d01_ref[lo:lo + _TILE, :] = jnp.concatenate(
            [d0.astype(jnp.int32), d1.astype(jnp.int32)], axis=1)
    ti = (jax.lax.broadcasted_iota(jnp.int32, (1, _NT), 1)
          .astype(jnp.float32) * _TILE)
    etf = jnp.zeros((1, _NT), jnp.float32)
    for e in range(_E):
        etf = etf + (ti >= base[:, e:e + 1]).astype(jnp.float32)
    et_ref[...] = etf.astype(jnp.int32) - 1


def _sc_dispatch(x, d0, d1):
    mesh = plsc.VectorSubcoreMesh(core_axis_name="c", subcore_axis_name="s")

    @functools.partial(
        pl.kernel, mesh=mesh,
        out_type=jax.ShapeDtypeStruct((_NP, _HID), jnp.float32),
        scratch_types=[
            pltpu.VMEM((32,), jnp.int32),
            pltpu.VMEM((32,), jnp.int32),
            pltpu.VMEM((32,), jnp.int32),
            pltpu.VMEM((32,), jnp.int32),
            pltpu.VMEM((32, _HID), jnp.float32),
            pltpu.VMEM((32, _HID), jnp.float32),
            pltpu.SemaphoreType.DMA,
            pltpu.SemaphoreType.DMA,
            pltpu.SemaphoreType.DMA,
            pltpu.SemaphoreType.DMA,
        ],
    )
    def k(x_hbm, d0_hbm, d1_hbm, xs_hbm, de0, do0, de1, do1, rows0, rows1,
          s00, s01, s10, s11):
        wid = lax.axis_index("s") * 2 + lax.axis_index("c")
        tok0 = wid * 64
        de = (de0, de1)
        do = (do0, do1)
        rows = (rows0, rows1)
        sa = (s00, s10)
        sb = (s01, s11)
        copies = []
        for c in range(2):
            pltpu.sync_copy(d0_hbm.at[pl.ds(tok0 + 32 * c, 32)], de[c])
            pltpu.sync_copy(d1_hbm.at[pl.ds(tok0 + 32 * c, 32)], do[c])
            pltpu.sync_copy(x_hbm.at[pl.ds(tok0 + 32 * c, 32)], rows[c])
            copies.append(pltpu.async_copy(rows[c], xs_hbm.at[de[c]], sa[c]))
            copies.append(pltpu.async_copy(rows[c], xs_hbm.at[do[c]], sb[c]))
        for cp in copies:
            cp.wait()

    return k(x, d0, d1)


def _gmm_body(et_ref, xs_ref, wg_ref, wu_ref, avg_ref, pk_ref):
    i = pl.program_id(0)
    e = et_ref[i]
    x = xs_ref[...]
    wg = wg_ref[pl.ds(e, 1)][0]
    wu = wu_ref[pl.ds(e, 1)][0]
    g = jax.lax.dot_general(
        x, wg, (((1,), (1,)), ((), ())), preferred_element_type=jnp.float32)
    u = jax.lax.dot_general(
        x, wu, (((1,), (1,)), ((), ())), preferred_element_type=jnp.float32)
    p = jnp.abs(u * g * jax.nn.sigmoid(g))
    n = p.shape[0]
    s = jax.lax.dot_general(
        p, avg_ref[...], (((1,), (0,)), ((), ())),
        preferred_element_type=jnp.float32)
    jot = jax.lax.broadcasted_iota(jnp.int32, (n, _INNER), 1)
    r = jnp.zeros((n, _INNER), jnp.int32)
    for jp in range(_INNER):
        c = s[:, jp:jp + 1]
        r = r + ((c > s) | ((c == s) & (jp < jot))).astype(jnp.int32)
    pk = e * 4096 + jnp.sum(
        jnp.where(r < 4, jnp.left_shift(jot, 3 * r), 0),
        axis=1, keepdims=True)
    pk_ref[...] = jnp.broadcast_to(pk, (n, 128))


def _sc_gather_words(pk, d0, d1):
    mesh = plsc.VectorSubcoreMesh(core_axis_name="c", subcore_axis_name="s")
    tpw = _TOK // _NW

    @functools.partial(
        pl.kernel, mesh=mesh,
        out_type=(jax.ShapeDtypeStruct((_TOK, 128), jnp.int32),
                  jax.ShapeDtypeStruct((_TOK, 128), jnp.int32)),
        scratch_types=[
            pltpu.VMEM((tpw,), jnp.int32),
            pltpu.VMEM((tpw,), jnp.int32),
            pltpu.VMEM((tpw, 128), jnp.int32),
            pltpu.VMEM((tpw, 128), jnp.int32),
            pltpu.SemaphoreType.DMA,
            pltpu.SemaphoreType.DMA,
        ],
    )
    def k(pk_hbm, d0_hbm, d1_hbm, o0_hbm, o1_hbm, i0, i1, r0, r1, s0, s1):
        wid = lax.axis_index("s") * 2 + lax.axis_index("c")
        base = wid * tpw
        pltpu.sync_copy(d0_hbm.at[pl.ds(base, tpw)], i0)
        pltpu.sync_copy(d1_hbm.at[pl.ds(base, tpw)], i1)
        c0 = pltpu.async_copy(pk_hbm.at[i0], r0, s0)
        c1 = pltpu.async_copy(pk_hbm.at[i1], r1, s1)
        c0.wait()
        pltpu.sync_copy(r0, o0_hbm.at[pl.ds(base, tpw)])
        c1.wait()
        pltpu.sync_copy(r1, o1_hbm.at[pl.ds(base, tpw)])

    return k(pk, d0, d1)


def _select_body(p0_ref, p1_ref, w01_ref, fw_ref, fid_ref):
    pk0 = p0_ref[:, 0:1]
    pk1 = p1_ref[:, 0:1]
    w01 = w01_ref[...]
    n = pk0.shape[0]
    e0 = pk0 >> 12
    e1 = pk1 >> 12
    t = [(pk0 >> (3 * rr)) & 7 for rr in range(4)]
    for (a, b) in ((0, 1), (2, 3), (0, 2), (1, 3), (1, 2)):
        hi = jnp.maximum(t[a], t[b])
        lo = jnp.minimum(t[a], t[b])
        t[a], t[b] = hi, lo
    ids0 = [x + e0 * _INNER for x in t]
    s1a = pk1 & 7
    s1b = (pk1 >> 3) & 7
    ids1 = [jnp.maximum(s1a, s1b) + e1 * _INNER,
            jnp.minimum(s1a, s1b) + e1 * _INNER]
    ids_a = jnp.concatenate(ids0 + ids1, axis=1)
    ids_b = jnp.concatenate(ids1 + ids0, axis=1)
    fid_ref[...] = jnp.where(e0 > e1, ids_a, ids_b)
    w0 = w01[:, 0:1]
    w1 = w01[:, 1:2]
    fw_ref[...] = jnp.concatenate(
        [jnp.broadcast_to(w0, (n, 4)), jnp.broadcast_to(w1, (n, 2))], axis=1)


def kernel(hidden_states, gating_output, topk, renormalize, out_gate_weight,
           stacked_in_gate_weights, stacked_in_up_weights):
    del gating_output, topk, renormalize
    x = hidden_states.astype(jnp.float32)
    n = x.shape[0]

    w01, d01, et2 = _pallas_call(
        _router_body,
        out_shape=(
            jax.ShapeDtypeStruct((n, _E), jnp.float32),
            jax.ShapeDtypeStruct((n, 2), jnp.int32),
            jax.ShapeDtypeStruct((1, _NT), jnp.int32),
        ),
    )(x, out_gate_weight.astype(jnp.float32))

    d0 = d01[:, 0].reshape(-1)
    d1 = d01[:, 1].reshape(-1)
    xs = _sc_dispatch(x, d0, d1)

    avg = (jnp.equal(
        jnp.arange(_OUT)[:, None] // _BSZ,
        jnp.arange(_INNER)[None, :]).astype(jnp.float32) / _BSZ)

    pk = _pallas_call(
        _gmm_body,
        grid_spec=pltpu.PrefetchScalarGridSpec(
            num_scalar_prefetch=1,
            grid=(_NT,),
            in_specs=[
                pl.BlockSpec((_TILE, _HID), lambda i, et: (i, 0)),
                pl.BlockSpec((_E, _OUT, _HID), lambda i, et: (0, 0, 0)),
                pl.BlockSpec((_E, _OUT, _HID), lambda i, et: (0, 0, 0)),
                pl.BlockSpec((_OUT, _INNER), lambda i, et: (0, 0)),
            ],
            out_specs=pl.BlockSpec((_TILE, 128), lambda i, et: (i, 0)),
        ),
        out_shape=jax.ShapeDtypeStruct((_NP, 128), jnp.int32),
    )(et2.reshape(-1), xs, stacked_in_gate_weights, stacked_in_up_weights, avg)

    p0, p1 = _sc_gather_words(pk, d0, d1)

    fw, fid = _pallas_call(
        _select_body,
        out_shape=(
            jax.ShapeDtypeStruct((n, 6), jnp.float32),
            jax.ShapeDtypeStruct((n, 6), jnp.int32),
        ),
    )(p0, p1, w01)
    return fw, fid
